# baseline (speedup 1.0000x reference)
import functools

import jax
import jax.numpy as jnp
from jax import lax
from jax.experimental import pallas as pl
from jax.experimental.pallas import tpu as pltpu
from jax.experimental.pallas import tpu_sc as plsc

N = 50000
E = 800000
B = 128

_LANES = 128
_RPAD = 6400
_RPT = _RPAD // 16
_RPT_HALF = _RPAD // 32
_NSPL = 10112
_NACC = 10120
_NOUT = 5 * _NSPL
_ZROWS = _NSPL // 16
_NDEG = 50048
_DROWS = _NDEG // 16
_F = 16
_EPT = _RPAD * _LANES // 16
_STEP = 256
_QB = 4

_f32 = jnp.float32
_i32 = jnp.int32


def _mesh():
    return plsc.VectorSubcoreMesh(core_axis_name="c", subcore_axis_name="s")


def _make_deg_kernel():
    @functools.partial(
        pl.kernel,
        out_type=(
            jax.ShapeDtypeStruct((_NDEG,), _f32),
            jax.ShapeDtypeStruct((_NDEG,), _f32),
        ),
        mesh=_mesh(),
        compiler_params=pltpu.CompilerParams(use_tc_tiling_on_sc=False,
                                             needs_layout_passes=False),
        scratch_types=[
            pltpu.VMEM((_RPT_HALF, _LANES), _i32),
            pltpu.VMEM((_LANES,), _f32),
            pltpu.VMEM((_DROWS,), _f32),
            pltpu.VMEM_SHARED((_NDEG,), _f32),
        ],
    )
    def deg_kernel(dst_hbm, zeros_hbm, ones_hbm, deg0_hbm, deg1_hbm,
                   dstv, onesv, zbuf, acc):
        cid = lax.axis_index("c")
        sid = lax.axis_index("s")
        row0 = sid * _DROWS
        pltpu.sync_copy(ones_hbm, onesv)
        pltpu.sync_copy(zeros_hbm, zbuf)
        pltpu.sync_copy(zbuf, acc.at[pl.ds(row0, _DROWS)])
        plsc.subcore_barrier()
        base = cid * (_RPT_HALF * 16) + sid * _RPT_HALF
        pltpu.sync_copy(dst_hbm.at[pl.ds(base, _RPT_HALF)], dstv)

        def body(r, carry):
            pltpu.sync_copy(onesv, acc.at[dstv.at[r]], add=True)
            return carry

        lax.fori_loop(0, _RPT_HALF, body, 0)
        plsc.subcore_barrier()

        @pl.when(cid == 0)
        def _():
            pltpu.sync_copy(acc.at[pl.ds(row0, _DROWS)],
                            deg0_hbm.at[pl.ds(row0, _DROWS)])

        @pl.when(cid == 1)
        def _():
            pltpu.sync_copy(acc.at[pl.ds(row0, _DROWS)],
                            deg1_hbm.at[pl.ds(row0, _DROWS)])

    return deg_kernel


def _make_agg_kernel(K):
    out_type = tuple(jax.ShapeDtypeStruct((_NOUT, _F), _f32)
                     for _ in range(K))
    scratch = [
        pltpu.VMEM((_EPT,), _i32),
        pltpu.VMEM((_EPT,), _i32),
        pltpu.VMEM((_QB * _STEP, _F), _f32),
        pltpu.VMEM_SHARED((_NACC, _F), _f32),
    ] + [pltpu.SemaphoreType.DMA] * _QB

    @functools.partial(
        pl.kernel, out_type=out_type, mesh=_mesh(),
        compiler_params=pltpu.CompilerParams(use_tc_tiling_on_sc=False,
                                             needs_layout_passes=False),
        scratch_types=scratch)
    def agg_kernel(*refs):
        src_hbm, dst_hbm, zeros_hbm = refs[0], refs[1], refs[2]
        tables = refs[3:3 + K]
        outs = refs[3 + K:3 + 2 * K]
        clsrc, cldst, rbuf, acc = refs[3 + 2 * K:3 + 2 * K + 4]
        sems = refs[3 + 2 * K + 4:]

        cid = lax.axis_index("c")
        sid = lax.axis_index("s")
        row0 = sid * _ZROWS
        ebase = sid * _EPT

        def split_body(s, carry):
            lo = pl.multiple_of(s * _NSPL, _NSPL)

            pltpu.sync_copy(src_hbm.at[pl.ds(ebase, _EPT)], clsrc)
            pltpu.sync_copy(dst_hbm.at[pl.ds(ebase, _EPT)], cldst)

            def scan_block(g, cursor):
                off = pl.multiple_of(g * 16, 16)
                sv = clsrc[pl.ds(off, 16)]
                dv = cldst[pl.ds(off, 16)]
                m = (dv >= lo) & (dv < lo + _NSPL)
                mi = m.astype(_i32)
                p = cursor + plsc.cumsum(mi) - 1
                plsc.store_scatter(clsrc, [p], sv, mask=m)
                plsc.store_scatter(cldst, [p], dv - lo, mask=m)
                return cursor + jnp.sum(mi)

            n = lax.fori_loop(0, _EPT // 16, scan_block, jnp.int32(0))

            blk = _QB * _STEP
            nup = (n + blk - 1) & ~jnp.int32(blk - 1)

            def pad_block(c, carry2):
                idx = lax.iota(_i32, 16) + c * 16 + n
                m = idx < nup
                plsc.store_scatter(clsrc, [idx], jnp.zeros((16,), _i32),
                                   mask=m)
                plsc.store_scatter(cldst, [idx],
                                   jnp.full((16,), _NSPL, _i32), mask=m)
                return carry2

            lax.fori_loop(0, blk // 16, pad_block, 0)
            nblk = nup // blk

            for k in range(K):
                own = k % 2

                @pl.when(cid == own)
                def _(k=k):
                    pltpu.sync_copy(zeros_hbm, acc.at[pl.ds(row0, _ZROWS)])

                plsc.subcore_barrier()

                @pl.when(cid == own)
                def _(k=k):
                    table = tables[k]

                    def body(jb, carry2):
                        cps = []
                        for q in range(_QB):
                            off = pl.multiple_of(
                                jb * _QB * _STEP + q * _STEP, _STEP)
                            cps.append((off, pltpu.async_copy(
                                table.at[clsrc.at[pl.ds(off, _STEP)]],
                                rbuf.at[pl.ds(q * _STEP, _STEP)], sems[q])))
                        for q in range(_QB):
                            off, cp = cps[q]
                            cp.wait()
                            pltpu.sync_copy(
                                rbuf.at[pl.ds(q * _STEP, _STEP)],
                                acc.at[cldst.at[pl.ds(off, _STEP)]],
                                add=True)
                        return carry2

                    lax.fori_loop(0, nblk, body, 0)

                plsc.subcore_barrier()

                @pl.when(cid == own)
                def _(k=k):
                    orow = pl.multiple_of(lo + row0, 8)
                    pltpu.sync_copy(acc.at[pl.ds(row0, _ZROWS)],
                                    outs[k].at[pl.ds(orow, _ZROWS)])

            return carry

        lax.fori_loop(0, 5, split_body, 0)

    return agg_kernel


_BN = 1000


def _full(spec_shape):
    return pl.BlockSpec(spec_shape, lambda i: (0,) * len(spec_shape))


def _rows(width):
    return pl.BlockSpec((_BN, width), lambda i: (i, 0))


def _combine(s, u, dv):
    return jnp.concatenate(
        [s[k][...] + u[k][...] for k in range(len(s))], axis=1) * dv[...]


def _chunk_out(outs, v):
    for k in range(len(outs)):
        outs[k][...] = v[:, _F * k:_F * (k + 1)]


def _tc_prep(x, deg0, deg1):
    def body(x_ref, d0_ref, d1_ref, dinv_ref, *outs):
        deg = d0_ref[...] + d1_ref[...] + 1.0
        dinv = lax.rsqrt(deg)
        dinv_ref[...] = dinv
        u = jnp.concatenate(
            [x_ref[...] * dinv, jnp.zeros((_BN, 6), _f32)], axis=1)
        _chunk_out(outs, u)

    return pl.pallas_call(
        body,
        grid=(N // _BN,),
        in_specs=[_rows(42), _rows(1), _rows(1)],
        out_specs=[_rows(1)] + [_rows(_F)] * 3,
        out_shape=[jax.ShapeDtypeStruct((N, 1), _f32)] +
                  [jax.ShapeDtypeStruct((N, _F), _f32)] * 3,
    )(x, deg0, deg1)


def _tc_layer1(s1, u1, dinv, W1p, b1):
    def body(*refs):
        s, u = refs[0:3], refs[3:6]
        dv, w, b = refs[6:9]
        outs = refs[9:16]
        t = _combine(s, u, dv)
        y = jnp.maximum(t @ w[...] + b[...], 0.0)
        up = jnp.concatenate(
            [y * dv[...], jnp.zeros((_BN, 12), _f32)], axis=1)
        _chunk_out(outs, up)

    return pl.pallas_call(
        body,
        grid=(N // _BN,),
        in_specs=[_rows(_F)] * 6 + [_rows(1), _full((48, 100)),
                                    _full((1, 100))],
        out_specs=[_rows(_F)] * 7,
        out_shape=[jax.ShapeDtypeStruct((N, _F), _f32)] * 7,
    )(*s1, *u1, dinv, W1p, b1)


def _tc_layer2(s2, u2, dinv, W2p, b2, W3):
    def body(*refs):
        s, u = refs[0:7], refs[7:14]
        dv, w2, b, w3 = refs[14:18]
        outs = refs[18:31]
        t = _combine(s, u, dv)
        y2 = jnp.maximum(t @ w2[...] + b[...], 0.0)
        u3 = (y2 @ w3[...]) * dv[...]
        up = jnp.concatenate([u3, jnp.zeros((_BN, 8), _f32)], axis=1)
        _chunk_out(outs, up)

    return pl.pallas_call(
        body,
        grid=(N // _BN,),
        in_specs=[_rows(_F)] * 14 + [_rows(1), _full((112, 400)),
                                     _full((1, 400)), _full((400, 200))],
        out_specs=[_rows(_F)] * 13,
        out_shape=[jax.ShapeDtypeStruct((N, _F), _f32)] * 13,
    )(*s2, *u2, dinv, W2p, b2, W3)


def _tc_layer3(s3, u3, dinv, b3, W4):
    def body(*refs):
        s, u = refs[0:13], refs[13:26]
        dv, b, w4 = refs[26:29]
        outs = refs[29:36]
        t = _combine(s, u, dv)
        y3 = jnp.maximum(t[:, :200] + b[...], 0.0)
        z4 = (y3 @ w4[...]) * dv[...]
        up = jnp.concatenate([z4, jnp.zeros((_BN, 12), _f32)], axis=1)
        _chunk_out(outs, up)

    return pl.pallas_call(
        body,
        grid=(N // _BN,),
        in_specs=[_rows(_F)] * 26 + [_rows(1), _full((1, 200)),
                                     _full((200, 100))],
        out_specs=[_rows(_F)] * 7,
        out_shape=[jax.ShapeDtypeStruct((N, _F), _f32)] * 7,
    )(*s3, *u3, dinv, b3, W4)


def _tc_final(s4, u4, dinv, b4, batch2d, Wo, bo, Wo2, bo2):
    ng = N // _BN

    def body(*refs):
        s, u = refs[0:7], refs[7:14]
        dv, b4r, bidx, wo, bor, wo2, bo2r = refs[14:21]
        out_ref = refs[21]
        gacc = refs[22]
        i = pl.program_id(0)
        t = _combine(s, u, dv)
        y4 = jnp.maximum(t[:, :100] + b4r[...], 0.0)
        y4a = jnp.concatenate(
            [y4, jnp.zeros((_BN, 27), _f32), jnp.ones((_BN, 1), _f32)],
            axis=1)
        lane = lax.broadcasted_iota(_i32, (_BN, 128), 1)
        onehot = (bidx[...] == lane).astype(_f32)
        part = lax.dot_general(onehot, y4a, (((0,), (0,)), ((), ())))

        @pl.when(i == 0)
        def _():
            gacc[...] = part

        @pl.when(i > 0)
        def _():
            gacc[...] += part

        @pl.when(i == ng - 1)
        def _():
            cnt = jnp.maximum(gacc[:, 127:128], 1.0)
            g = gacc[:, :100] / cnt
            o = jnp.maximum(g @ wo[...] + bor[...], 0.0)
            out_ref[...] = o @ wo2[...] + bo2r[...]

    return pl.pallas_call(
        body,
        grid=(ng,),
        in_specs=[_rows(_F)] * 14 + [
            _rows(1), _full((1, 100)), _rows(1),
            _full((100, 50)), _full((1, 50)), _full((50, 1)), _full((1, 1))],
        out_specs=pl.BlockSpec((B, 1), lambda i: (0, 0)),
        out_shape=jax.ShapeDtypeStruct((B, 1), _f32),
        scratch_shapes=[pltpu.VMEM((B, 128), _f32)],
    )(*s4, *u4, dinv, b4, batch2d, Wo, bo, Wo2, bo2)


def kernel(x, edge_index, batch_index, W1, b1, W2, b2, W3, b3, W4, b4,
           Wo, bo, Wo2, bo2):
    src = edge_index[0]
    dst = edge_index[1]
    pad = _RPAD * _LANES - E
    src_f = jnp.concatenate([src, jnp.zeros((pad,), _i32)])
    dst_f = jnp.concatenate([dst, jnp.full((pad,), N, _i32)])
    src_r = src_f.reshape(_RPAD, _LANES)
    dst_r = dst_f.reshape(_RPAD, _LANES)

    zeros_deg = jnp.zeros((_DROWS,), _f32)
    ones_deg = jnp.ones((_LANES,), _f32)
    zeros16 = jnp.zeros((_ZROWS, _F), _f32)

    deg0, deg1 = _make_deg_kernel()(dst_r, zeros_deg, ones_deg)
    deg0 = deg0[:N].reshape(N, 1)
    deg1 = deg1[:N].reshape(N, 1)

    W1p = jnp.concatenate([W1, jnp.zeros((6, 100), _f32)], axis=0)
    W2p = jnp.concatenate([W2, jnp.zeros((12, 400), _f32)], axis=0)
    batch2d = batch_index.reshape(N, 1)

    dinv, *u1 = _tc_prep(x, deg0, deg1)

    s1 = _make_agg_kernel(3)(src_f, dst_f, zeros16, *u1)
    u2 = _tc_layer1(s1, u1, dinv, W1p, b1.reshape(1, 100))

    s2 = _make_agg_kernel(7)(src_f, dst_f, zeros16, *u2)
    u3 = _tc_layer2(s2, u2, dinv, W2p, b2.reshape(1, 400), W3)

    s3 = _make_agg_kernel(13)(src_f, dst_f, zeros16, *u3)
    u4 = _tc_layer3(s3, u3, dinv, b3.reshape(1, 200), W4)

    s4 = _make_agg_kernel(7)(src_f, dst_f, zeros16, *u4)
    return _tc_final(s4, u4, dinv, b4.reshape(1, 100), batch2d,
                     Wo, bo.reshape(1, 50), Wo2, bo2.reshape(1, 1))


# speedup vs baseline: 1.0236x; 1.0236x over previous
import functools

import jax
import jax.numpy as jnp
from jax import lax
from jax.experimental import pallas as pl
from jax.experimental.pallas import tpu as pltpu
from jax.experimental.pallas import tpu_sc as plsc

N = 50000
E = 800000
B = 128

_LANES = 128
_RPAD = 6400
_RPT = _RPAD // 16
_RPT_HALF = _RPAD // 32
_NSPL = 10112
_NACC = 10120
_NOUT = 5 * _NSPL
_ZROWS = _NSPL // 16
_NDEG = 50048
_DROWS = _NDEG // 16
_F = 16
_EPT = _RPAD * _LANES // 16
_STEP = 256
_QB = 4

_f32 = jnp.float32
_i32 = jnp.int32


def _mesh():
    return plsc.VectorSubcoreMesh(core_axis_name="c", subcore_axis_name="s")


def _make_deg_kernel():
    @functools.partial(
        pl.kernel,
        out_type=(
            jax.ShapeDtypeStruct((_NDEG,), _f32),
            jax.ShapeDtypeStruct((_NDEG,), _f32),
        ),
        mesh=_mesh(),
        compiler_params=pltpu.CompilerParams(use_tc_tiling_on_sc=False,
                                             needs_layout_passes=False),
        scratch_types=[
            pltpu.VMEM((_RPT_HALF, _LANES), _i32),
            pltpu.VMEM((_LANES,), _f32),
            pltpu.VMEM((_DROWS,), _f32),
            pltpu.VMEM_SHARED((_NDEG,), _f32),
        ],
    )
    def deg_kernel(dst_hbm, zeros_hbm, ones_hbm, deg0_hbm, deg1_hbm,
                   dstv, onesv, zbuf, acc):
        cid = lax.axis_index("c")
        sid = lax.axis_index("s")
        row0 = sid * _DROWS
        pltpu.sync_copy(ones_hbm, onesv)
        pltpu.sync_copy(zeros_hbm, zbuf)
        pltpu.sync_copy(zbuf, acc.at[pl.ds(row0, _DROWS)])
        plsc.subcore_barrier()
        base = cid * (_RPT_HALF * 16) + sid * _RPT_HALF
        pltpu.sync_copy(dst_hbm.at[pl.ds(base, _RPT_HALF)], dstv)

        def body(r, carry):
            pltpu.sync_copy(onesv, acc.at[dstv.at[r]], add=True)
            return carry

        lax.fori_loop(0, _RPT_HALF, body, 0)
        plsc.subcore_barrier()

        @pl.when(cid == 0)
        def _():
            pltpu.sync_copy(acc.at[pl.ds(row0, _DROWS)],
                            deg0_hbm.at[pl.ds(row0, _DROWS)])

        @pl.when(cid == 1)
        def _():
            pltpu.sync_copy(acc.at[pl.ds(row0, _DROWS)],
                            deg1_hbm.at[pl.ds(row0, _DROWS)])

    return deg_kernel


def _make_agg_kernel(K):
    out_type = tuple(jax.ShapeDtypeStruct((_NOUT, _F), _f32)
                     for _ in range(K))
    scratch = [
        pltpu.VMEM((_EPT,), _i32),
        pltpu.VMEM((_EPT,), _i32),
        pltpu.VMEM((_QB * _STEP, _F), _f32),
        pltpu.VMEM_SHARED((_NACC, _F), _f32),
    ] + [pltpu.SemaphoreType.DMA] * _QB

    @functools.partial(
        pl.kernel, out_type=out_type, mesh=_mesh(),
        compiler_params=pltpu.CompilerParams(use_tc_tiling_on_sc=False,
                                             needs_layout_passes=False),
        scratch_types=scratch)
    def agg_kernel(*refs):
        src_hbm, dst_hbm, zeros_hbm = refs[0], refs[1], refs[2]
        tables = refs[3:3 + K]
        outs = refs[3 + K:3 + 2 * K]
        clsrc, cldst, rbuf, acc = refs[3 + 2 * K:3 + 2 * K + 4]
        sems = refs[3 + 2 * K + 4:]

        cid = lax.axis_index("c")
        sid = lax.axis_index("s")
        row0 = sid * _ZROWS
        ebase = sid * _EPT

        def split_body(s, carry):
            lo = pl.multiple_of(s * _NSPL, _NSPL)

            pltpu.sync_copy(src_hbm.at[pl.ds(ebase, _EPT)], clsrc)
            pltpu.sync_copy(dst_hbm.at[pl.ds(ebase, _EPT)], cldst)

            def scan_block(g, cursor):
                off = pl.multiple_of(g * 64, 64)
                svs, dvs, ms, pcs = [], [], [], []
                for j in range(4):
                    sv = clsrc[pl.ds(off + j * 16, 16)]
                    dv = cldst[pl.ds(off + j * 16, 16)]
                    m = (dv >= lo) & (dv < lo + _NSPL)
                    svs.append(sv)
                    dvs.append(dv)
                    ms.append(m)
                    pcs.append(plsc.cumsum(m.astype(_i32)))
                for j in range(4):
                    p = cursor + pcs[j] - 1
                    plsc.store_scatter(clsrc, [p], svs[j], mask=ms[j])
                    plsc.store_scatter(cldst, [p], dvs[j] - lo, mask=ms[j])
                    cursor = cursor + pcs[j][15]
                return cursor

            n = lax.fori_loop(0, _EPT // 64, scan_block, jnp.int32(0))

            blk = _QB * _STEP
            nup = (n + blk - 1) & ~jnp.int32(blk - 1)

            def pad_block(c, carry2):
                idx = lax.iota(_i32, 16) + c * 16 + n
                m = idx < nup
                plsc.store_scatter(clsrc, [idx], jnp.zeros((16,), _i32),
                                   mask=m)
                plsc.store_scatter(cldst, [idx],
                                   jnp.full((16,), _NSPL, _i32), mask=m)
                return carry2

            lax.fori_loop(0, blk // 16, pad_block, 0)
            nblk = nup // blk

            for k in range(K):
                own = k % 2

                @pl.when(cid == own)
                def _(k=k):
                    pltpu.sync_copy(zeros_hbm, acc.at[pl.ds(row0, _ZROWS)])

                plsc.subcore_barrier()

                @pl.when(cid == own)
                def _(k=k):
                    table = tables[k]

                    def body(jb, carry2):
                        cps = []
                        for q in range(_QB):
                            off = pl.multiple_of(
                                jb * _QB * _STEP + q * _STEP, _STEP)
                            cps.append((off, pltpu.async_copy(
                                table.at[clsrc.at[pl.ds(off, _STEP)]],
                                rbuf.at[pl.ds(q * _STEP, _STEP)], sems[q])))
                        for q in range(_QB):
                            off, cp = cps[q]
                            cp.wait()
                            pltpu.sync_copy(
                                rbuf.at[pl.ds(q * _STEP, _STEP)],
                                acc.at[cldst.at[pl.ds(off, _STEP)]],
                                add=True)
                        return carry2

                    lax.fori_loop(0, nblk, body, 0)

                plsc.subcore_barrier()

                @pl.when(cid == own)
                def _(k=k):
                    orow = pl.multiple_of(lo + row0, 8)
                    pltpu.sync_copy(acc.at[pl.ds(row0, _ZROWS)],
                                    outs[k].at[pl.ds(orow, _ZROWS)])

            return carry

        lax.fori_loop(0, 5, split_body, 0)

    return agg_kernel


_BN = 1000


def _full(spec_shape):
    return pl.BlockSpec(spec_shape, lambda i: (0,) * len(spec_shape))


def _rows(width):
    return pl.BlockSpec((_BN, width), lambda i: (i, 0))


def _combine(s, u, dv):
    return jnp.concatenate(
        [s[k][...] + u[k][...] for k in range(len(s))], axis=1) * dv[...]


def _chunk_out(outs, v):
    for k in range(len(outs)):
        outs[k][...] = v[:, _F * k:_F * (k + 1)]


def _tc_prep(x, deg0, deg1):
    def body(x_ref, d0_ref, d1_ref, dinv_ref, *outs):
        deg = d0_ref[...] + d1_ref[...] + 1.0
        dinv = lax.rsqrt(deg)
        dinv_ref[...] = dinv
        u = jnp.concatenate(
            [x_ref[...] * dinv, jnp.zeros((_BN, 6), _f32)], axis=1)
        _chunk_out(outs, u)

    return pl.pallas_call(
        body,
        grid=(N // _BN,),
        in_specs=[_rows(42), _rows(1), _rows(1)],
        out_specs=[_rows(1)] + [_rows(_F)] * 3,
        out_shape=[jax.ShapeDtypeStruct((N, 1), _f32)] +
                  [jax.ShapeDtypeStruct((N, _F), _f32)] * 3,
    )(x, deg0, deg1)


def _tc_layer1(s1, u1, dinv, W1p, b1):
    def body(*refs):
        s, u = refs[0:3], refs[3:6]
        dv, w, b = refs[6:9]
        outs = refs[9:16]
        t = _combine(s, u, dv)
        y = jnp.maximum(t @ w[...] + b[...], 0.0)
        up = jnp.concatenate(
            [y * dv[...], jnp.zeros((_BN, 12), _f32)], axis=1)
        _chunk_out(outs, up)

    return pl.pallas_call(
        body,
        grid=(N // _BN,),
        in_specs=[_rows(_F)] * 6 + [_rows(1), _full((48, 100)),
                                    _full((1, 100))],
        out_specs=[_rows(_F)] * 7,
        out_shape=[jax.ShapeDtypeStruct((N, _F), _f32)] * 7,
    )(*s1, *u1, dinv, W1p, b1)


def _tc_layer2(s2, u2, dinv, W2p, b2, W3):
    def body(*refs):
        s, u = refs[0:7], refs[7:14]
        dv, w2, b, w3 = refs[14:18]
        outs = refs[18:31]
        t = _combine(s, u, dv)
        y2 = jnp.maximum(t @ w2[...] + b[...], 0.0)
        u3 = (y2 @ w3[...]) * dv[...]
        up = jnp.concatenate([u3, jnp.zeros((_BN, 8), _f32)], axis=1)
        _chunk_out(outs, up)

    return pl.pallas_call(
        body,
        grid=(N // _BN,),
        in_specs=[_rows(_F)] * 14 + [_rows(1), _full((112, 400)),
                                     _full((1, 400)), _full((400, 200))],
        out_specs=[_rows(_F)] * 13,
        out_shape=[jax.ShapeDtypeStruct((N, _F), _f32)] * 13,
    )(*s2, *u2, dinv, W2p, b2, W3)


def _tc_layer3(s3, u3, dinv, b3, W4):
    def body(*refs):
        s, u = refs[0:13], refs[13:26]
        dv, b, w4 = refs[26:29]
        outs = refs[29:36]
        t = _combine(s, u, dv)
        y3 = jnp.maximum(t[:, :200] + b[...], 0.0)
        z4 = (y3 @ w4[...]) * dv[...]
        up = jnp.concatenate([z4, jnp.zeros((_BN, 12), _f32)], axis=1)
        _chunk_out(outs, up)

    return pl.pallas_call(
        body,
        grid=(N // _BN,),
        in_specs=[_rows(_F)] * 26 + [_rows(1), _full((1, 200)),
                                     _full((200, 100))],
        out_specs=[_rows(_F)] * 7,
        out_shape=[jax.ShapeDtypeStruct((N, _F), _f32)] * 7,
    )(*s3, *u3, dinv, b3, W4)


def _tc_final(s4, u4, dinv, b4, batch2d, Wo, bo, Wo2, bo2):
    ng = N // _BN

    def body(*refs):
        s, u = refs[0:7], refs[7:14]
        dv, b4r, bidx, wo, bor, wo2, bo2r = refs[14:21]
        out_ref = refs[21]
        gacc = refs[22]
        i = pl.program_id(0)
        t = _combine(s, u, dv)
        y4 = jnp.maximum(t[:, :100] + b4r[...], 0.0)
        y4a = jnp.concatenate(
            [y4, jnp.zeros((_BN, 27), _f32), jnp.ones((_BN, 1), _f32)],
            axis=1)
        lane = lax.broadcasted_iota(_i32, (_BN, 128), 1)
        onehot = (bidx[...] == lane).astype(_f32)
        part = lax.dot_general(onehot, y4a, (((0,), (0,)), ((), ())))

        @pl.when(i == 0)
        def _():
            gacc[...] = part

        @pl.when(i > 0)
        def _():
            gacc[...] += part

        @pl.when(i == ng - 1)
        def _():
            cnt = jnp.maximum(gacc[:, 127:128], 1.0)
            g = gacc[:, :100] / cnt
            o = jnp.maximum(g @ wo[...] + bor[...], 0.0)
            out_ref[...] = o @ wo2[...] + bo2r[...]

    return pl.pallas_call(
        body,
        grid=(ng,),
        in_specs=[_rows(_F)] * 14 + [
            _rows(1), _full((1, 100)), _rows(1),
            _full((100, 50)), _full((1, 50)), _full((50, 1)), _full((1, 1))],
        out_specs=pl.BlockSpec((B, 1), lambda i: (0, 0)),
        out_shape=jax.ShapeDtypeStruct((B, 1), _f32),
        scratch_shapes=[pltpu.VMEM((B, 128), _f32)],
    )(*s4, *u4, dinv, b4, batch2d, Wo, bo, Wo2, bo2)


def kernel(x, edge_index, batch_index, W1, b1, W2, b2, W3, b3, W4, b4,
           Wo, bo, Wo2, bo2):
    src = edge_index[0]
    dst = edge_index[1]
    pad = _RPAD * _LANES - E
    src_f = jnp.concatenate([src, jnp.zeros((pad,), _i32)])
    dst_f = jnp.concatenate([dst, jnp.full((pad,), N, _i32)])
    src_r = src_f.reshape(_RPAD, _LANES)
    dst_r = dst_f.reshape(_RPAD, _LANES)

    zeros_deg = jnp.zeros((_DROWS,), _f32)
    ones_deg = jnp.ones((_LANES,), _f32)
    zeros16 = jnp.zeros((_ZROWS, _F), _f32)

    deg0, deg1 = _make_deg_kernel()(dst_r, zeros_deg, ones_deg)
    deg0 = deg0[:N].reshape(N, 1)
    deg1 = deg1[:N].reshape(N, 1)

    W1p = jnp.concatenate([W1, jnp.zeros((6, 100), _f32)], axis=0)
    W2p = jnp.concatenate([W2, jnp.zeros((12, 400), _f32)], axis=0)
    batch2d = batch_index.reshape(N, 1)

    dinv, *u1 = _tc_prep(x, deg0, deg1)

    s1 = _make_agg_kernel(3)(src_f, dst_f, zeros16, *u1)
    u2 = _tc_layer1(s1, u1, dinv, W1p, b1.reshape(1, 100))

    s2 = _make_agg_kernel(7)(src_f, dst_f, zeros16, *u2)
    u3 = _tc_layer2(s2, u2, dinv, W2p, b2.reshape(1, 400), W3)

    s3 = _make_agg_kernel(13)(src_f, dst_f, zeros16, *u3)
    u4 = _tc_layer3(s3, u3, dinv, b3.reshape(1, 200), W4)

    s4 = _make_agg_kernel(7)(src_f, dst_f, zeros16, *u4)
    return _tc_final(s4, u4, dinv, b4.reshape(1, 100), batch2d,
                     Wo, bo.reshape(1, 50), Wo2, bo2.reshape(1, 1))


# speedup vs baseline: 1.1269x; 1.1009x over previous
import functools

import jax
import jax.numpy as jnp
from jax import lax
from jax.experimental import pallas as pl
from jax.experimental.pallas import tpu as pltpu
from jax.experimental.pallas import tpu_sc as plsc

N = 50000
E = 800000
B = 128

_LANES = 128
_RPAD = 6400
_RPT = _RPAD // 16
_RPT_HALF = _RPAD // 32
_NSPL = 5632
_NACC = 5640
_NOUT = 9 * _NSPL
_ZROWS = _NSPL // 16
_NDEG = 50048
_DROWS = _NDEG // 16
_F = 32
_EPT = _RPAD * _LANES // 16
_STEP = 128
_QB = 4

_f32 = jnp.float32
_i32 = jnp.int32


def _mesh():
    return plsc.VectorSubcoreMesh(core_axis_name="c", subcore_axis_name="s")


def _make_deg_kernel():
    @functools.partial(
        pl.kernel,
        out_type=(
            jax.ShapeDtypeStruct((_NDEG,), _f32),
            jax.ShapeDtypeStruct((_NDEG,), _f32),
        ),
        mesh=_mesh(),
        compiler_params=pltpu.CompilerParams(use_tc_tiling_on_sc=False,
                                             needs_layout_passes=False),
        scratch_types=[
            pltpu.VMEM((_RPT_HALF, _LANES), _i32),
            pltpu.VMEM((_LANES,), _f32),
            pltpu.VMEM((_DROWS,), _f32),
            pltpu.VMEM_SHARED((_NDEG,), _f32),
        ],
    )
    def deg_kernel(dst_hbm, zeros_hbm, ones_hbm, deg0_hbm, deg1_hbm,
                   dstv, onesv, zbuf, acc):
        cid = lax.axis_index("c")
        sid = lax.axis_index("s")
        row0 = sid * _DROWS
        pltpu.sync_copy(ones_hbm, onesv)
        pltpu.sync_copy(zeros_hbm, zbuf)
        pltpu.sync_copy(zbuf, acc.at[pl.ds(row0, _DROWS)])
        plsc.subcore_barrier()
        base = cid * (_RPT_HALF * 16) + sid * _RPT_HALF
        pltpu.sync_copy(dst_hbm.at[pl.ds(base, _RPT_HALF)], dstv)

        def body(r, carry):
            pltpu.sync_copy(onesv, acc.at[dstv.at[r]], add=True)
            return carry

        lax.fori_loop(0, _RPT_HALF, body, 0)
        plsc.subcore_barrier()

        @pl.when(cid == 0)
        def _():
            pltpu.sync_copy(acc.at[pl.ds(row0, _DROWS)],
                            deg0_hbm.at[pl.ds(row0, _DROWS)])

        @pl.when(cid == 1)
        def _():
            pltpu.sync_copy(acc.at[pl.ds(row0, _DROWS)],
                            deg1_hbm.at[pl.ds(row0, _DROWS)])

    return deg_kernel


def _make_agg_kernel(K):
    out_type = tuple(jax.ShapeDtypeStruct((_NOUT, _F), _f32)
                     for _ in range(K))
    scratch = [
        pltpu.VMEM((_EPT,), _i32),
        pltpu.VMEM((_EPT,), _i32),
        pltpu.VMEM((_QB * _STEP, _F), _f32),
        pltpu.VMEM_SHARED((_NACC, _F), _f32),
    ] + [pltpu.SemaphoreType.DMA] * _QB

    @functools.partial(
        pl.kernel, out_type=out_type, mesh=_mesh(),
        compiler_params=pltpu.CompilerParams(use_tc_tiling_on_sc=False,
                                             needs_layout_passes=False),
        scratch_types=scratch)
    def agg_kernel(*refs):
        src_hbm, dst_hbm, zeros_hbm = refs[0], refs[1], refs[2]
        tables = refs[3:3 + K]
        outs = refs[3 + K:3 + 2 * K]
        clsrc, cldst, rbuf, acc = refs[3 + 2 * K:3 + 2 * K + 4]
        sems = refs[3 + 2 * K + 4:]

        cid = lax.axis_index("c")
        sid = lax.axis_index("s")
        row0 = sid * _ZROWS
        ebase = sid * _EPT

        def split_body(s, carry):
            lo = pl.multiple_of(s * _NSPL, _NSPL)

            pltpu.sync_copy(src_hbm.at[pl.ds(ebase, _EPT)], clsrc)
            pltpu.sync_copy(dst_hbm.at[pl.ds(ebase, _EPT)], cldst)

            def scan_block(g, cursor):
                off = pl.multiple_of(g * 64, 64)
                svs, dvs, ms, pcs = [], [], [], []
                for j in range(4):
                    sv = clsrc[pl.ds(off + j * 16, 16)]
                    dv = cldst[pl.ds(off + j * 16, 16)]
                    m = (dv >= lo) & (dv < lo + _NSPL)
                    svs.append(sv)
                    dvs.append(dv)
                    ms.append(m)
                    pcs.append(plsc.cumsum(m.astype(_i32)))
                for j in range(4):
                    p = cursor + pcs[j] - 1
                    plsc.store_scatter(clsrc, [p], svs[j], mask=ms[j])
                    plsc.store_scatter(cldst, [p], dvs[j] - lo, mask=ms[j])
                    cursor = cursor + pcs[j][15]
                return cursor

            n = lax.fori_loop(0, _EPT // 64, scan_block, jnp.int32(0))

            blk = _QB * _STEP
            nup = (n + blk - 1) & ~jnp.int32(blk - 1)

            def pad_block(c, carry2):
                idx = lax.iota(_i32, 16) + c * 16 + n
                m = idx < nup
                plsc.store_scatter(clsrc, [idx], jnp.zeros((16,), _i32),
                                   mask=m)
                plsc.store_scatter(cldst, [idx],
                                   jnp.full((16,), _NSPL, _i32), mask=m)
                return carry2

            lax.fori_loop(0, blk // 16, pad_block, 0)
            nblk = nup // blk

            for k in range(K):
                own = k % 2

                @pl.when(cid == own)
                def _(k=k):
                    pltpu.sync_copy(zeros_hbm, acc.at[pl.ds(row0, _ZROWS)])

                plsc.subcore_barrier()

                @pl.when(cid == own)
                def _(k=k):
                    table = tables[k]

                    def body(jb, carry2):
                        cps = []
                        for q in range(_QB):
                            off = pl.multiple_of(
                                jb * _QB * _STEP + q * _STEP, _STEP)
                            cps.append((off, pltpu.async_copy(
                                table.at[clsrc.at[pl.ds(off, _STEP)]],
                                rbuf.at[pl.ds(q * _STEP, _STEP)], sems[q])))
                        for q in range(_QB):
                            off, cp = cps[q]
                            cp.wait()
                            pltpu.sync_copy(
                                rbuf.at[pl.ds(q * _STEP, _STEP)],
                                acc.at[cldst.at[pl.ds(off, _STEP)]],
                                add=True)
                        return carry2

                    lax.fori_loop(0, nblk, body, 0)

                plsc.subcore_barrier()

                @pl.when(cid == own)
                def _(k=k):
                    orow = pl.multiple_of(lo + row0, 8)
                    pltpu.sync_copy(acc.at[pl.ds(row0, _ZROWS)],
                                    outs[k].at[pl.ds(orow, _ZROWS)])

            return carry

        lax.fori_loop(0, 9, split_body, 0)

    return agg_kernel


_BN = 1000


def _full(spec_shape):
    return pl.BlockSpec(spec_shape, lambda i: (0,) * len(spec_shape))


def _rows(width):
    return pl.BlockSpec((_BN, width), lambda i: (i, 0))


def _combine(s, u, dv):
    return jnp.concatenate(
        [s[k][...] + u[k][...] for k in range(len(s))], axis=1) * dv[...]


def _chunk_out(outs, v):
    for k in range(len(outs)):
        outs[k][...] = v[:, _F * k:_F * (k + 1)]


def _tc_prep(x, deg0, deg1):
    def body(x_ref, d0_ref, d1_ref, dinv_ref, *outs):
        deg = d0_ref[...] + d1_ref[...] + 1.0
        dinv = lax.rsqrt(deg)
        dinv_ref[...] = dinv
        u = jnp.concatenate(
            [x_ref[...] * dinv, jnp.zeros((_BN, 22), _f32)], axis=1)
        _chunk_out(outs, u)

    return pl.pallas_call(
        body,
        grid=(N // _BN,),
        in_specs=[_rows(42), _rows(1), _rows(1)],
        out_specs=[_rows(1)] + [_rows(_F)] * 2,
        out_shape=[jax.ShapeDtypeStruct((N, 1), _f32)] +
                  [jax.ShapeDtypeStruct((N, _F), _f32)] * 2,
    )(x, deg0, deg1)


def _tc_layer1(s1, u1, dinv, W1p, b1):
    def body(*refs):
        s, u = refs[0:2], refs[2:4]
        dv, w, b = refs[4:7]
        outs = refs[7:11]
        t = _combine(s, u, dv)
        y = jnp.maximum(t @ w[...] + b[...], 0.0)
        up = jnp.concatenate(
            [y * dv[...], jnp.zeros((_BN, 28), _f32)], axis=1)
        _chunk_out(outs, up)

    return pl.pallas_call(
        body,
        grid=(N // _BN,),
        in_specs=[_rows(_F)] * 4 + [_rows(1), _full((64, 100)),
                                    _full((1, 100))],
        out_specs=[_rows(_F)] * 4,
        out_shape=[jax.ShapeDtypeStruct((N, _F), _f32)] * 4,
    )(*s1, *u1, dinv, W1p, b1)


def _tc_layer2(s2, u2, dinv, W2p, b2, W3):
    def body(*refs):
        s, u = refs[0:4], refs[4:8]
        dv, w2, b, w3 = refs[8:12]
        outs = refs[12:19]
        t = _combine(s, u, dv)
        y2 = jnp.maximum(t @ w2[...] + b[...], 0.0)
        u3 = (y2 @ w3[...]) * dv[...]
        up = jnp.concatenate([u3, jnp.zeros((_BN, 24), _f32)], axis=1)
        _chunk_out(outs, up)

    return pl.pallas_call(
        body,
        grid=(N // _BN,),
        in_specs=[_rows(_F)] * 8 + [_rows(1), _full((128, 400)),
                                    _full((1, 400)), _full((400, 200))],
        out_specs=[_rows(_F)] * 7,
        out_shape=[jax.ShapeDtypeStruct((N, _F), _f32)] * 7,
    )(*s2, *u2, dinv, W2p, b2, W3)


def _tc_layer3(s3, u3, dinv, b3, W4):
    def body(*refs):
        s, u = refs[0:7], refs[7:14]
        dv, b, w4 = refs[14:17]
        outs = refs[17:21]
        t = _combine(s, u, dv)
        y3 = jnp.maximum(t[:, :200] + b[...], 0.0)
        z4 = (y3 @ w4[...]) * dv[...]
        up = jnp.concatenate([z4, jnp.zeros((_BN, 28), _f32)], axis=1)
        _chunk_out(outs, up)

    return pl.pallas_call(
        body,
        grid=(N // _BN,),
        in_specs=[_rows(_F)] * 14 + [_rows(1), _full((1, 200)),
                                     _full((200, 100))],
        out_specs=[_rows(_F)] * 4,
        out_shape=[jax.ShapeDtypeStruct((N, _F), _f32)] * 4,
    )(*s3, *u3, dinv, b3, W4)


def _tc_final(s4, u4, dinv, b4, batch2d, Wo, bo, Wo2, bo2):
    ng = N // _BN

    def body(*refs):
        s, u = refs[0:4], refs[4:8]
        dv, b4r, bidx, wo, bor, wo2, bo2r = refs[8:15]
        out_ref = refs[15]
        gacc = refs[16]
        i = pl.program_id(0)
        t = _combine(s, u, dv)
        y4 = jnp.maximum(t[:, :100] + b4r[...], 0.0)
        y4a = jnp.concatenate(
            [y4, jnp.zeros((_BN, 27), _f32), jnp.ones((_BN, 1), _f32)],
            axis=1)
        lane = lax.broadcasted_iota(_i32, (_BN, 128), 1)
        onehot = (bidx[...] == lane).astype(_f32)
        part = lax.dot_general(onehot, y4a, (((0,), (0,)), ((), ())))

        @pl.when(i == 0)
        def _():
            gacc[...] = part

        @pl.when(i > 0)
        def _():
            gacc[...] += part

        @pl.when(i == ng - 1)
        def _():
            cnt = jnp.maximum(gacc[:, 127:128], 1.0)
            g = gacc[:, :100] / cnt
            o = jnp.maximum(g @ wo[...] + bor[...], 0.0)
            out_ref[...] = o @ wo2[...] + bo2r[...]

    return pl.pallas_call(
        body,
        grid=(ng,),
        in_specs=[_rows(_F)] * 8 + [
            _rows(1), _full((1, 100)), _rows(1),
            _full((100, 50)), _full((1, 50)), _full((50, 1)), _full((1, 1))],
        out_specs=pl.BlockSpec((B, 1), lambda i: (0, 0)),
        out_shape=jax.ShapeDtypeStruct((B, 1), _f32),
        scratch_shapes=[pltpu.VMEM((B, 128), _f32)],
    )(*s4, *u4, dinv, b4, batch2d, Wo, bo, Wo2, bo2)


def kernel(x, edge_index, batch_index, W1, b1, W2, b2, W3, b3, W4, b4,
           Wo, bo, Wo2, bo2):
    src = edge_index[0]
    dst = edge_index[1]
    pad = _RPAD * _LANES - E
    src_f = jnp.concatenate([src, jnp.zeros((pad,), _i32)])
    dst_f = jnp.concatenate([dst, jnp.full((pad,), N, _i32)])
    src_r = src_f.reshape(_RPAD, _LANES)
    dst_r = dst_f.reshape(_RPAD, _LANES)

    zeros_deg = jnp.zeros((_DROWS,), _f32)
    ones_deg = jnp.ones((_LANES,), _f32)
    zerosf = jnp.zeros((_ZROWS, _F), _f32)

    deg0, deg1 = _make_deg_kernel()(dst_r, zeros_deg, ones_deg)
    deg0 = deg0[:N].reshape(N, 1)
    deg1 = deg1[:N].reshape(N, 1)

    W1p = jnp.concatenate([W1, jnp.zeros((22, 100), _f32)], axis=0)
    W2p = jnp.concatenate([W2, jnp.zeros((28, 400), _f32)], axis=0)
    batch2d = batch_index.reshape(N, 1)

    dinv, *u1 = _tc_prep(x, deg0, deg1)

    s1 = _make_agg_kernel(2)(src_f, dst_f, zerosf, *u1)
    u2 = _tc_layer1(s1, u1, dinv, W1p, b1.reshape(1, 100))

    s2 = _make_agg_kernel(4)(src_f, dst_f, zerosf, *u2)
    u3 = _tc_layer2(s2, u2, dinv, W2p, b2.reshape(1, 400), W3)

    s3 = _make_agg_kernel(7)(src_f, dst_f, zerosf, *u3)
    u4 = _tc_layer3(s3, u3, dinv, b3.reshape(1, 200), W4)

    s4 = _make_agg_kernel(4)(src_f, dst_f, zerosf, *u4)
    return _tc_final(s4, u4, dinv, b4.reshape(1, 100), batch2d,
                     Wo, bo.reshape(1, 50), Wo2, bo2.reshape(1, 1))


# speedup vs baseline: 1.1534x; 1.0235x over previous
import functools

import jax
import jax.numpy as jnp
from jax import lax
from jax.experimental import pallas as pl
from jax.experimental.pallas import tpu as pltpu
from jax.experimental.pallas import tpu_sc as plsc

N = 50000
E = 800000
B = 128

_LANES = 128
_RPAD = 6400
_RPT = _RPAD // 16
_RPT_HALF = _RPAD // 32
_NSPL = 5632
_NACC = 5640
_NOUT = 9 * _NSPL
_ZROWS = _NSPL // 16
_NDEG = 50048
_DROWS = _NDEG // 16
_F = 32
_EPT = _RPAD * _LANES // 16
_STEP = 128
_QB = 4

_f32 = jnp.float32
_i32 = jnp.int32


def _mesh():
    return plsc.VectorSubcoreMesh(core_axis_name="c", subcore_axis_name="s")


def _make_deg_kernel():
    @functools.partial(
        pl.kernel,
        out_type=(
            jax.ShapeDtypeStruct((_NDEG,), _f32),
            jax.ShapeDtypeStruct((_NDEG,), _f32),
        ),
        mesh=_mesh(),
        compiler_params=pltpu.CompilerParams(use_tc_tiling_on_sc=False,
                                             needs_layout_passes=False),
        scratch_types=[
            pltpu.VMEM((_RPT_HALF, _LANES), _i32),
            pltpu.VMEM((_LANES,), _f32),
            pltpu.VMEM((_DROWS,), _f32),
            pltpu.VMEM_SHARED((_NDEG,), _f32),
        ],
    )
    def deg_kernel(dst_hbm, zeros_hbm, ones_hbm, deg0_hbm, deg1_hbm,
                   dstv, onesv, zbuf, acc):
        cid = lax.axis_index("c")
        sid = lax.axis_index("s")
        row0 = sid * _DROWS
        pltpu.sync_copy(ones_hbm, onesv)
        pltpu.sync_copy(zeros_hbm, zbuf)
        pltpu.sync_copy(zbuf, acc.at[pl.ds(row0, _DROWS)])
        plsc.subcore_barrier()
        base = cid * (_RPT_HALF * 16) + sid * _RPT_HALF
        pltpu.sync_copy(dst_hbm.at[pl.ds(base, _RPT_HALF)], dstv)

        def body(r, carry):
            pltpu.sync_copy(onesv, acc.at[dstv.at[r]], add=True)
            return carry

        lax.fori_loop(0, _RPT_HALF, body, 0)
        plsc.subcore_barrier()

        @pl.when(cid == 0)
        def _():
            pltpu.sync_copy(acc.at[pl.ds(row0, _DROWS)],
                            deg0_hbm.at[pl.ds(row0, _DROWS)])

        @pl.when(cid == 1)
        def _():
            pltpu.sync_copy(acc.at[pl.ds(row0, _DROWS)],
                            deg1_hbm.at[pl.ds(row0, _DROWS)])

    return deg_kernel


def _make_agg_kernel(K):
    out_type = tuple(jax.ShapeDtypeStruct((_NOUT, _F), _f32)
                     for _ in range(K))
    scratch = [
        pltpu.VMEM((_EPT,), _i32),
        pltpu.VMEM((_EPT,), _i32),
        pltpu.VMEM((_QB * _STEP, _F), _f32),
        pltpu.VMEM_SHARED((_NACC, _F), _f32),
    ] + [pltpu.SemaphoreType.DMA] * _QB

    @functools.partial(
        pl.kernel, out_type=out_type, mesh=_mesh(),
        compiler_params=pltpu.CompilerParams(use_tc_tiling_on_sc=False,
                                             needs_layout_passes=False),
        scratch_types=scratch)
    def agg_kernel(*refs):
        src_hbm, dst_hbm, zeros_hbm = refs[0], refs[1], refs[2]
        tables = refs[3:3 + K]
        outs = refs[3 + K:3 + 2 * K]
        clsrc, cldst, rbuf, acc = refs[3 + 2 * K:3 + 2 * K + 4]
        sems = refs[3 + 2 * K + 4:]

        cid = lax.axis_index("c")
        sid = lax.axis_index("s")
        row0 = sid * _ZROWS
        ebase = sid * _EPT

        def split_body(s, carry):
            lo = pl.multiple_of(s * _NSPL, _NSPL)

            pltpu.sync_copy(src_hbm.at[pl.ds(ebase, _EPT)], clsrc)
            pltpu.sync_copy(dst_hbm.at[pl.ds(ebase, _EPT)], cldst)

            def scan_block(g, cursor):
                off = pl.multiple_of(g * 64, 64)
                svs, dvs, ms, pcs = [], [], [], []
                for j in range(4):
                    sv = clsrc[pl.ds(off + j * 16, 16)]
                    dv = cldst[pl.ds(off + j * 16, 16)]
                    m = (dv >= lo) & (dv < lo + _NSPL)
                    svs.append(sv)
                    dvs.append(dv)
                    ms.append(m)
                    pcs.append(plsc.cumsum(m.astype(_i32)))
                for j in range(4):
                    p = cursor + pcs[j] - 1
                    plsc.store_scatter(clsrc, [p], svs[j], mask=ms[j])
                    plsc.store_scatter(cldst, [p], dvs[j] - lo, mask=ms[j])
                    cursor = cursor + pcs[j][15]
                return cursor

            n = lax.fori_loop(0, _EPT // 64, scan_block, jnp.int32(0))

            blk = _QB * _STEP
            nup = (n + blk - 1) & ~jnp.int32(blk - 1)

            def pad_block(c, carry2):
                idx = lax.iota(_i32, 16) + c * 16 + n
                m = idx < nup
                plsc.store_scatter(clsrc, [idx], jnp.zeros((16,), _i32),
                                   mask=m)
                plsc.store_scatter(cldst, [idx],
                                   jnp.full((16,), _NSPL, _i32), mask=m)
                return carry2

            lax.fori_loop(0, blk // 16, pad_block, 0)
            nblk = nup // blk

            for k in range(K):
                own = (k + s) % 2

                @pl.when(cid == own)
                def _(k=k):
                    pltpu.sync_copy(zeros_hbm, acc.at[pl.ds(row0, _ZROWS)])

                plsc.subcore_barrier()

                @pl.when(cid == own)
                def _(k=k):
                    table = tables[k]

                    def body(jb, carry2):
                        cps = []
                        for q in range(_QB):
                            off = pl.multiple_of(
                                jb * _QB * _STEP + q * _STEP, _STEP)
                            cps.append((off, pltpu.async_copy(
                                table.at[clsrc.at[pl.ds(off, _STEP)]],
                                rbuf.at[pl.ds(q * _STEP, _STEP)], sems[q])))
                        for q in range(_QB):
                            off, cp = cps[q]
                            cp.wait()
                            pltpu.sync_copy(
                                rbuf.at[pl.ds(q * _STEP, _STEP)],
                                acc.at[cldst.at[pl.ds(off, _STEP)]],
                                add=True)
                        return carry2

                    lax.fori_loop(0, nblk, body, 0)

                plsc.subcore_barrier()

                @pl.when(cid == own)
                def _(k=k):
                    orow = pl.multiple_of(lo + row0, 8)
                    pltpu.sync_copy(acc.at[pl.ds(row0, _ZROWS)],
                                    outs[k].at[pl.ds(orow, _ZROWS)])

            return carry

        lax.fori_loop(0, 9, split_body, 0)

    return agg_kernel


_BN = 1000


def _full(spec_shape):
    return pl.BlockSpec(spec_shape, lambda i: (0,) * len(spec_shape))


def _rows(width):
    return pl.BlockSpec((_BN, width), lambda i: (i, 0))


def _combine(s, u, dv):
    return jnp.concatenate(
        [s[k][...] + u[k][...] for k in range(len(s))], axis=1) * dv[...]


def _chunk_out(outs, v):
    for k in range(len(outs)):
        outs[k][...] = v[:, _F * k:_F * (k + 1)]


def _tc_prep(x, deg0, deg1):
    def body(x_ref, d0_ref, d1_ref, dinv_ref, *outs):
        deg = d0_ref[...] + d1_ref[...] + 1.0
        dinv = lax.rsqrt(deg)
        dinv_ref[...] = dinv
        u = jnp.concatenate(
            [x_ref[...] * dinv, jnp.zeros((_BN, 22), _f32)], axis=1)
        _chunk_out(outs, u)

    return pl.pallas_call(
        body,
        grid=(N // _BN,),
        in_specs=[_rows(42), _rows(1), _rows(1)],
        out_specs=[_rows(1)] + [_rows(_F)] * 2,
        out_shape=[jax.ShapeDtypeStruct((N, 1), _f32)] +
                  [jax.ShapeDtypeStruct((N, _F), _f32)] * 2,
    )(x, deg0, deg1)


def _tc_layer1(s1, u1, dinv, W1p, b1):
    def body(*refs):
        s, u = refs[0:2], refs[2:4]
        dv, w, b = refs[4:7]
        outs = refs[7:11]
        t = _combine(s, u, dv)
        y = jnp.maximum(t @ w[...] + b[...], 0.0)
        up = jnp.concatenate(
            [y * dv[...], jnp.zeros((_BN, 28), _f32)], axis=1)
        _chunk_out(outs, up)

    return pl.pallas_call(
        body,
        grid=(N // _BN,),
        in_specs=[_rows(_F)] * 4 + [_rows(1), _full((64, 100)),
                                    _full((1, 100))],
        out_specs=[_rows(_F)] * 4,
        out_shape=[jax.ShapeDtypeStruct((N, _F), _f32)] * 4,
    )(*s1, *u1, dinv, W1p, b1)


def _tc_layer2(s2, u2, dinv, W2p, b2, W3):
    def body(*refs):
        s, u = refs[0:4], refs[4:8]
        dv, w2, b, w3 = refs[8:12]
        outs = refs[12:19]
        t = _combine(s, u, dv)
        y2 = jnp.maximum(t @ w2[...] + b[...], 0.0)
        u3 = (y2 @ w3[...]) * dv[...]
        up = jnp.concatenate([u3, jnp.zeros((_BN, 24), _f32)], axis=1)
        _chunk_out(outs, up)

    return pl.pallas_call(
        body,
        grid=(N // _BN,),
        in_specs=[_rows(_F)] * 8 + [_rows(1), _full((128, 400)),
                                    _full((1, 400)), _full((400, 200))],
        out_specs=[_rows(_F)] * 7,
        out_shape=[jax.ShapeDtypeStruct((N, _F), _f32)] * 7,
    )(*s2, *u2, dinv, W2p, b2, W3)


def _tc_layer3(s3, u3, dinv, b3, W4):
    def body(*refs):
        s, u = refs[0:7], refs[7:14]
        dv, b, w4 = refs[14:17]
        outs = refs[17:21]
        t = _combine(s, u, dv)
        y3 = jnp.maximum(t[:, :200] + b[...], 0.0)
        z4 = (y3 @ w4[...]) * dv[...]
        up = jnp.concatenate([z4, jnp.zeros((_BN, 28), _f32)], axis=1)
        _chunk_out(outs, up)

    return pl.pallas_call(
        body,
        grid=(N // _BN,),
        in_specs=[_rows(_F)] * 14 + [_rows(1), _full((1, 200)),
                                     _full((200, 100))],
        out_specs=[_rows(_F)] * 4,
        out_shape=[jax.ShapeDtypeStruct((N, _F), _f32)] * 4,
    )(*s3, *u3, dinv, b3, W4)


def _tc_final(s4, u4, dinv, b4, batch2d, Wo, bo, Wo2, bo2):
    ng = N // _BN

    def body(*refs):
        s, u = refs[0:4], refs[4:8]
        dv, b4r, bidx, wo, bor, wo2, bo2r = refs[8:15]
        out_ref = refs[15]
        gacc = refs[16]
        i = pl.program_id(0)
        t = _combine(s, u, dv)
        y4 = jnp.maximum(t[:, :100] + b4r[...], 0.0)
        y4a = jnp.concatenate(
            [y4, jnp.zeros((_BN, 27), _f32), jnp.ones((_BN, 1), _f32)],
            axis=1)
        lane = lax.broadcasted_iota(_i32, (_BN, 128), 1)
        onehot = (bidx[...] == lane).astype(_f32)
        part = lax.dot_general(onehot, y4a, (((0,), (0,)), ((), ())))

        @pl.when(i == 0)
        def _():
            gacc[...] = part

        @pl.when(i > 0)
        def _():
            gacc[...] += part

        @pl.when(i == ng - 1)
        def _():
            cnt = jnp.maximum(gacc[:, 127:128], 1.0)
            g = gacc[:, :100] / cnt
            o = jnp.maximum(g @ wo[...] + bor[...], 0.0)
            out_ref[...] = o @ wo2[...] + bo2r[...]

    return pl.pallas_call(
        body,
        grid=(ng,),
        in_specs=[_rows(_F)] * 8 + [
            _rows(1), _full((1, 100)), _rows(1),
            _full((100, 50)), _full((1, 50)), _full((50, 1)), _full((1, 1))],
        out_specs=pl.BlockSpec((B, 1), lambda i: (0, 0)),
        out_shape=jax.ShapeDtypeStruct((B, 1), _f32),
        scratch_shapes=[pltpu.VMEM((B, 128), _f32)],
    )(*s4, *u4, dinv, b4, batch2d, Wo, bo, Wo2, bo2)


def kernel(x, edge_index, batch_index, W1, b1, W2, b2, W3, b3, W4, b4,
           Wo, bo, Wo2, bo2):
    src = edge_index[0]
    dst = edge_index[1]
    pad = _RPAD * _LANES - E
    src_f = jnp.concatenate([src, jnp.zeros((pad,), _i32)])
    dst_f = jnp.concatenate([dst, jnp.full((pad,), N, _i32)])
    src_r = src_f.reshape(_RPAD, _LANES)
    dst_r = dst_f.reshape(_RPAD, _LANES)

    zeros_deg = jnp.zeros((_DROWS,), _f32)
    ones_deg = jnp.ones((_LANES,), _f32)
    zerosf = jnp.zeros((_ZROWS, _F), _f32)

    deg0, deg1 = _make_deg_kernel()(dst_r, zeros_deg, ones_deg)
    deg0 = deg0[:N].reshape(N, 1)
    deg1 = deg1[:N].reshape(N, 1)

    W1p = jnp.concatenate([W1, jnp.zeros((22, 100), _f32)], axis=0)
    W2p = jnp.concatenate([W2, jnp.zeros((28, 400), _f32)], axis=0)
    batch2d = batch_index.reshape(N, 1)

    dinv, *u1 = _tc_prep(x, deg0, deg1)

    s1 = _make_agg_kernel(2)(src_f, dst_f, zerosf, *u1)
    u2 = _tc_layer1(s1, u1, dinv, W1p, b1.reshape(1, 100))

    s2 = _make_agg_kernel(4)(src_f, dst_f, zerosf, *u2)
    u3 = _tc_layer2(s2, u2, dinv, W2p, b2.reshape(1, 400), W3)

    s3 = _make_agg_kernel(7)(src_f, dst_f, zerosf, *u3)
    u4 = _tc_layer3(s3, u3, dinv, b3.reshape(1, 200), W4)

    s4 = _make_agg_kernel(4)(src_f, dst_f, zerosf, *u4)
    return _tc_final(s4, u4, dinv, b4.reshape(1, 100), batch2d,
                     Wo, bo.reshape(1, 50), Wo2, bo2.reshape(1, 1))


# speedup vs baseline: 1.1558x; 1.0021x over previous
import functools

import jax
import jax.numpy as jnp
from jax import lax
from jax.experimental import pallas as pl
from jax.experimental.pallas import tpu as pltpu
from jax.experimental.pallas import tpu_sc as plsc

N = 50000
E = 800000
B = 128

_LANES = 128
_RPAD = 6400
_RPT = _RPAD // 16
_RPT_HALF = _RPAD // 32
_NSPL = 5632
_NACC = 5640
_NOUT = 9 * _NSPL
_ZROWS = _NSPL // 16
_NDEG = 50048
_DROWS = _NDEG // 16
_F = 32
_EPT = _RPAD * _LANES // 16
_STEP = 256
_QB = 2

_f32 = jnp.float32
_i32 = jnp.int32


def _mesh():
    return plsc.VectorSubcoreMesh(core_axis_name="c", subcore_axis_name="s")


def _make_deg_kernel():
    @functools.partial(
        pl.kernel,
        out_type=(
            jax.ShapeDtypeStruct((_NDEG,), _f32),
            jax.ShapeDtypeStruct((_NDEG,), _f32),
        ),
        mesh=_mesh(),
        compiler_params=pltpu.CompilerParams(use_tc_tiling_on_sc=False,
                                             needs_layout_passes=False),
        scratch_types=[
            pltpu.VMEM((_RPT_HALF, _LANES), _i32),
            pltpu.VMEM((_LANES,), _f32),
            pltpu.VMEM((_DROWS,), _f32),
            pltpu.VMEM_SHARED((_NDEG,), _f32),
        ],
    )
    def deg_kernel(dst_hbm, zeros_hbm, ones_hbm, deg0_hbm, deg1_hbm,
                   dstv, onesv, zbuf, acc):
        cid = lax.axis_index("c")
        sid = lax.axis_index("s")
        row0 = sid * _DROWS
        pltpu.sync_copy(ones_hbm, onesv)
        pltpu.sync_copy(zeros_hbm, zbuf)
        pltpu.sync_copy(zbuf, acc.at[pl.ds(row0, _DROWS)])
        plsc.subcore_barrier()
        base = cid * (_RPT_HALF * 16) + sid * _RPT_HALF
        pltpu.sync_copy(dst_hbm.at[pl.ds(base, _RPT_HALF)], dstv)

        def body(r, carry):
            pltpu.sync_copy(onesv, acc.at[dstv.at[r]], add=True)
            return carry

        lax.fori_loop(0, _RPT_HALF, body, 0)
        plsc.subcore_barrier()

        @pl.when(cid == 0)
        def _():
            pltpu.sync_copy(acc.at[pl.ds(row0, _DROWS)],
                            deg0_hbm.at[pl.ds(row0, _DROWS)])

        @pl.when(cid == 1)
        def _():
            pltpu.sync_copy(acc.at[pl.ds(row0, _DROWS)],
                            deg1_hbm.at[pl.ds(row0, _DROWS)])

    return deg_kernel


def _make_agg_kernel(K):
    out_type = tuple(jax.ShapeDtypeStruct((_NOUT, _F), _f32)
                     for _ in range(K))
    scratch = [
        pltpu.VMEM((_EPT,), _i32),
        pltpu.VMEM((_EPT,), _i32),
        pltpu.VMEM((_QB * _STEP, _F), _f32),
        pltpu.VMEM_SHARED((_NACC, _F), _f32),
    ] + [pltpu.SemaphoreType.DMA] * _QB

    @functools.partial(
        pl.kernel, out_type=out_type, mesh=_mesh(),
        compiler_params=pltpu.CompilerParams(use_tc_tiling_on_sc=False,
                                             needs_layout_passes=False),
        scratch_types=scratch)
    def agg_kernel(*refs):
        src_hbm, dst_hbm, zeros_hbm = refs[0], refs[1], refs[2]
        tables = refs[3:3 + K]
        outs = refs[3 + K:3 + 2 * K]
        clsrc, cldst, rbuf, acc = refs[3 + 2 * K:3 + 2 * K + 4]
        sems = refs[3 + 2 * K + 4:]

        cid = lax.axis_index("c")
        sid = lax.axis_index("s")
        row0 = sid * _ZROWS
        ebase = sid * _EPT

        def split_body(s, carry):
            lo = pl.multiple_of(s * _NSPL, _NSPL)

            pltpu.sync_copy(src_hbm.at[pl.ds(ebase, _EPT)], clsrc)
            pltpu.sync_copy(dst_hbm.at[pl.ds(ebase, _EPT)], cldst)

            def scan_block(g, cursor):
                off = pl.multiple_of(g * 64, 64)
                svs, dvs, ms, pcs = [], [], [], []
                for j in range(4):
                    sv = clsrc[pl.ds(off + j * 16, 16)]
                    dv = cldst[pl.ds(off + j * 16, 16)]
                    m = (dv >= lo) & (dv < lo + _NSPL)
                    svs.append(sv)
                    dvs.append(dv)
                    ms.append(m)
                    pcs.append(plsc.cumsum(m.astype(_i32)))
                for j in range(4):
                    p = cursor + pcs[j] - 1
                    plsc.store_scatter(clsrc, [p], svs[j], mask=ms[j])
                    plsc.store_scatter(cldst, [p], dvs[j] - lo, mask=ms[j])
                    cursor = cursor + pcs[j][15]
                return cursor

            n = lax.fori_loop(0, _EPT // 64, scan_block, jnp.int32(0))

            blk = _QB * _STEP
            nup = (n + blk - 1) & ~jnp.int32(blk - 1)

            def pad_block(c, carry2):
                idx = lax.iota(_i32, 16) + c * 16 + n
                m = idx < nup
                plsc.store_scatter(clsrc, [idx], jnp.zeros((16,), _i32),
                                   mask=m)
                plsc.store_scatter(cldst, [idx],
                                   jnp.full((16,), _NSPL, _i32), mask=m)
                return carry2

            lax.fori_loop(0, blk // 16, pad_block, 0)
            nblk = nup // blk

            for k in range(K):
                own = (k + s) % 2

                @pl.when(cid == own)
                def _(k=k):
                    pltpu.sync_copy(zeros_hbm, acc.at[pl.ds(row0, _ZROWS)])

                plsc.subcore_barrier()

                @pl.when(cid == own)
                def _(k=k):
                    table = tables[k]

                    def body(jb, carry2):
                        cps = []
                        for q in range(_QB):
                            off = pl.multiple_of(
                                jb * _QB * _STEP + q * _STEP, _STEP)
                            cps.append((off, pltpu.async_copy(
                                table.at[clsrc.at[pl.ds(off, _STEP)]],
                                rbuf.at[pl.ds(q * _STEP, _STEP)], sems[q])))
                        for q in range(_QB):
                            off, cp = cps[q]
                            cp.wait()
                            pltpu.sync_copy(
                                rbuf.at[pl.ds(q * _STEP, _STEP)],
                                acc.at[cldst.at[pl.ds(off, _STEP)]],
                                add=True)
                        return carry2

                    lax.fori_loop(0, nblk, body, 0)

                plsc.subcore_barrier()

                @pl.when(cid == own)
                def _(k=k):
                    orow = pl.multiple_of(lo + row0, 8)
                    pltpu.sync_copy(acc.at[pl.ds(row0, _ZROWS)],
                                    outs[k].at[pl.ds(orow, _ZROWS)])

            return carry

        lax.fori_loop(0, 9, split_body, 0)

    return agg_kernel


_BN = 1000


def _full(spec_shape):
    return pl.BlockSpec(spec_shape, lambda i: (0,) * len(spec_shape))


def _rows(width):
    return pl.BlockSpec((_BN, width), lambda i: (i, 0))


def _combine(s, u, dv):
    return jnp.concatenate(
        [s[k][...] + u[k][...] for k in range(len(s))], axis=1) * dv[...]


def _chunk_out(outs, v):
    for k in range(len(outs)):
        outs[k][...] = v[:, _F * k:_F * (k + 1)]


def _tc_prep(x, deg0, deg1):
    def body(x_ref, d0_ref, d1_ref, dinv_ref, *outs):
        deg = d0_ref[...] + d1_ref[...] + 1.0
        dinv = lax.rsqrt(deg)
        dinv_ref[...] = dinv
        u = jnp.concatenate(
            [x_ref[...] * dinv, jnp.zeros((_BN, 22), _f32)], axis=1)
        _chunk_out(outs, u)

    return pl.pallas_call(
        body,
        grid=(N // _BN,),
        in_specs=[_rows(42), _rows(1), _rows(1)],
        out_specs=[_rows(1)] + [_rows(_F)] * 2,
        out_shape=[jax.ShapeDtypeStruct((N, 1), _f32)] +
                  [jax.ShapeDtypeStruct((N, _F), _f32)] * 2,
    )(x, deg0, deg1)


def _tc_layer1(s1, u1, dinv, W1p, b1):
    def body(*refs):
        s, u = refs[0:2], refs[2:4]
        dv, w, b = refs[4:7]
        outs = refs[7:11]
        t = _combine(s, u, dv)
        y = jnp.maximum(t @ w[...] + b[...], 0.0)
        up = jnp.concatenate(
            [y * dv[...], jnp.zeros((_BN, 28), _f32)], axis=1)
        _chunk_out(outs, up)

    return pl.pallas_call(
        body,
        grid=(N // _BN,),
        in_specs=[_rows(_F)] * 4 + [_rows(1), _full((64, 100)),
                                    _full((1, 100))],
        out_specs=[_rows(_F)] * 4,
        out_shape=[jax.ShapeDtypeStruct((N, _F), _f32)] * 4,
    )(*s1, *u1, dinv, W1p, b1)


def _tc_layer2(s2, u2, dinv, W2p, b2, W3):
    def body(*refs):
        s, u = refs[0:4], refs[4:8]
        dv, w2, b, w3 = refs[8:12]
        outs = refs[12:19]
        t = _combine(s, u, dv)
        y2 = jnp.maximum(t @ w2[...] + b[...], 0.0)
        u3 = (y2 @ w3[...]) * dv[...]
        up = jnp.concatenate([u3, jnp.zeros((_BN, 24), _f32)], axis=1)
        _chunk_out(outs, up)

    return pl.pallas_call(
        body,
        grid=(N // _BN,),
        in_specs=[_rows(_F)] * 8 + [_rows(1), _full((128, 400)),
                                    _full((1, 400)), _full((400, 200))],
        out_specs=[_rows(_F)] * 7,
        out_shape=[jax.ShapeDtypeStruct((N, _F), _f32)] * 7,
    )(*s2, *u2, dinv, W2p, b2, W3)


def _tc_layer3(s3, u3, dinv, b3, W4):
    def body(*refs):
        s, u = refs[0:7], refs[7:14]
        dv, b, w4 = refs[14:17]
        outs = refs[17:21]
        t = _combine(s, u, dv)
        y3 = jnp.maximum(t[:, :200] + b[...], 0.0)
        z4 = (y3 @ w4[...]) * dv[...]
        up = jnp.concatenate([z4, jnp.zeros((_BN, 28), _f32)], axis=1)
        _chunk_out(outs, up)

    return pl.pallas_call(
        body,
        grid=(N // _BN,),
        in_specs=[_rows(_F)] * 14 + [_rows(1), _full((1, 200)),
                                     _full((200, 100))],
        out_specs=[_rows(_F)] * 4,
        out_shape=[jax.ShapeDtypeStruct((N, _F), _f32)] * 4,
    )(*s3, *u3, dinv, b3, W4)


def _tc_final(s4, u4, dinv, b4, batch2d, Wo, bo, Wo2, bo2):
    ng = N // _BN

    def body(*refs):
        s, u = refs[0:4], refs[4:8]
        dv, b4r, bidx, wo, bor, wo2, bo2r = refs[8:15]
        out_ref = refs[15]
        gacc = refs[16]
        i = pl.program_id(0)
        t = _combine(s, u, dv)
        y4 = jnp.maximum(t[:, :100] + b4r[...], 0.0)
        y4a = jnp.concatenate(
            [y4, jnp.zeros((_BN, 27), _f32), jnp.ones((_BN, 1), _f32)],
            axis=1)
        lane = lax.broadcasted_iota(_i32, (_BN, 128), 1)
        onehot = (bidx[...] == lane).astype(_f32)
        part = lax.dot_general(onehot, y4a, (((0,), (0,)), ((), ())))

        @pl.when(i == 0)
        def _():
            gacc[...] = part

        @pl.when(i > 0)
        def _():
            gacc[...] += part

        @pl.when(i == ng - 1)
        def _():
            cnt = jnp.maximum(gacc[:, 127:128], 1.0)
            g = gacc[:, :100] / cnt
            o = jnp.maximum(g @ wo[...] + bor[...], 0.0)
            out_ref[...] = o @ wo2[...] + bo2r[...]

    return pl.pallas_call(
        body,
        grid=(ng,),
        in_specs=[_rows(_F)] * 8 + [
            _rows(1), _full((1, 100)), _rows(1),
            _full((100, 50)), _full((1, 50)), _full((50, 1)), _full((1, 1))],
        out_specs=pl.BlockSpec((B, 1), lambda i: (0, 0)),
        out_shape=jax.ShapeDtypeStruct((B, 1), _f32),
        scratch_shapes=[pltpu.VMEM((B, 128), _f32)],
    )(*s4, *u4, dinv, b4, batch2d, Wo, bo, Wo2, bo2)


def kernel(x, edge_index, batch_index, W1, b1, W2, b2, W3, b3, W4, b4,
           Wo, bo, Wo2, bo2):
    src = edge_index[0]
    dst = edge_index[1]
    pad = _RPAD * _LANES - E
    src_f = jnp.concatenate([src, jnp.zeros((pad,), _i32)])
    dst_f = jnp.concatenate([dst, jnp.full((pad,), N, _i32)])
    src_r = src_f.reshape(_RPAD, _LANES)
    dst_r = dst_f.reshape(_RPAD, _LANES)

    zeros_deg = jnp.zeros((_DROWS,), _f32)
    ones_deg = jnp.ones((_LANES,), _f32)
    zerosf = jnp.zeros((_ZROWS, _F), _f32)

    deg0, deg1 = _make_deg_kernel()(dst_r, zeros_deg, ones_deg)
    deg0 = deg0[:N].reshape(N, 1)
    deg1 = deg1[:N].reshape(N, 1)

    W1p = jnp.concatenate([W1, jnp.zeros((22, 100), _f32)], axis=0)
    W2p = jnp.concatenate([W2, jnp.zeros((28, 400), _f32)], axis=0)
    batch2d = batch_index.reshape(N, 1)

    dinv, *u1 = _tc_prep(x, deg0, deg1)

    s1 = _make_agg_kernel(2)(src_f, dst_f, zerosf, *u1)
    u2 = _tc_layer1(s1, u1, dinv, W1p, b1.reshape(1, 100))

    s2 = _make_agg_kernel(4)(src_f, dst_f, zerosf, *u2)
    u3 = _tc_layer2(s2, u2, dinv, W2p, b2.reshape(1, 400), W3)

    s3 = _make_agg_kernel(7)(src_f, dst_f, zerosf, *u3)
    u4 = _tc_layer3(s3, u3, dinv, b3.reshape(1, 200), W4)

    s4 = _make_agg_kernel(4)(src_f, dst_f, zerosf, *u4)
    return _tc_final(s4, u4, dinv, b4.reshape(1, 100), batch2d,
                     Wo, bo.reshape(1, 50), Wo2, bo2.reshape(1, 1))
